# baseline (speedup 1.0000x reference)
import functools

import jax
import jax.numpy as jnp
from jax import lax
from jax.experimental import pallas as pl
from jax.experimental.pallas import tpu as pltpu
from jax.experimental.pallas import tpu_sc as plsc


def _make_sc_gather(bsz, emb, n_workers, n_chunks, chunk):
    mesh = plsc.VectorSubcoreMesh(core_axis_name="c", subcore_axis_name="s")
    rows_per_w = n_chunks * chunk
    half = n_workers // 2
    assert rows_per_w * half == bsz

    nbuf = 6
    nl = 16

    @functools.partial(
        pl.kernel,
        mesh=mesh,
        out_type=jax.ShapeDtypeStruct((bsz, 2 * emb), jnp.float32),
        scratch_types=[
            pltpu.VMEM((rows_per_w * 3 + 16,), jnp.float32),
            pltpu.VMEM((n_chunks, chunk), jnp.int32),
            [pltpu.VMEM((chunk, emb), jnp.float32) for _ in range(nbuf)],
            [pltpu.SemaphoreType.DMA for _ in range(nbuf)],
            [pltpu.SemaphoreType.DMA for _ in range(nbuf)],
        ],
    )
    def sc_gather(ids_hbm, table_hbm, out_hbm, ids_v, idx_v, rows, gsems, ssems):
        wid = lax.axis_index("s") * 2 + lax.axis_index("c")
        base = (wid % half) * rows_per_w
        col_sel = wid // half
        col = col_sel * emb
        pltpu.sync_copy(
            ids_hbm.at[pl.ds(base * 3, rows_per_w * 3)],
            ids_v.at[pl.ds(0, rows_per_w * 3)])
        lane = lax.iota(jnp.int32, nl)

        def take16(v, il):
            return lax.gather(
                v, il[:, None],
                lax.GatherDimensionNumbers(
                    offset_dims=(), collapsed_slice_dims=(0,),
                    start_index_map=(0,)),
                (1,), mode=lax.GatherScatterMode.PROMISE_IN_BOUNDS)
        il0 = (lane * 3) & 15
        il1 = (lane * 3 - 16) & 15
        il2 = (lane * 3 - 32) & 15
        m0 = lane < 6
        m1 = jnp.logical_and(lane >= 6, lane < 11)
        for j in range(n_chunks):
            for k in range(chunk // nl):
                w = (j * chunk + k * nl) * 3 + col_sel
                v0 = ids_v[pl.ds(w, nl)]
                v1 = ids_v[pl.ds(w + nl, nl)]
                v2 = ids_v[pl.ds(w + 2 * nl, nl)]
                vals = jnp.where(
                    m0, take16(v0, il0),
                    jnp.where(m1, take16(v1, il1),
                              take16(v2, il2)))
                idx_v[j, pl.ds(k * nl, nl)] = vals.astype(jnp.int32)
        gh = [None] * nbuf
        sh = [None] * nbuf

        def store_dst(j):
            return out_hbm.at[pl.ds(base + j * chunk, chunk), pl.ds(col, emb)]

        for j in range(min(nbuf, n_chunks)):
            gh[j] = pltpu.async_copy(
                table_hbm.at[idx_v.at[j]], rows[j], gsems[j])
        for j in range(n_chunks):
            s = j % nbuf
            gh[s].wait()
            sh[s] = pltpu.async_copy(rows[s], store_dst(j), ssems[s])
            nxt = j - 1 + nbuf
            if j >= 1 and nxt < n_chunks:
                t = nxt % nbuf
                sh[t].wait()
                gh[t] = pltpu.async_copy(
                    table_hbm.at[idx_v.at[nxt]], rows[t], gsems[t])
        for j in range(max(0, n_chunks - nbuf), n_chunks):
            sh[j % nbuf].wait()

    return sc_gather


def _mlp_body(n_blocks, bb, feats_ref, sd_ref, w1ab_ref, w1c_ref, b1_ref,
              gamma_ref, beta_ref, w2_ref, b2_ref, out_ref, x_scratch):
    i = pl.program_id(0)
    xbt = lax.dot_general(
        w1ab_ref[:], feats_ref[:], (((1,), (1,)), ((), ())),
        preferred_element_type=jnp.float32,
    )
    x_scratch[:, pl.ds(i * bb, bb)] = (
        xbt + sd_ref[:] * w1c_ref[:] + b1_ref[:])

    @pl.when(i == n_blocks - 1)
    def _finish():
        x = x_scratch[:]
        mean = jnp.mean(x, axis=1, keepdims=True)
        var = jnp.mean((x - mean) ** 2, axis=1, keepdims=True)
        xn = (x - mean) * lax.rsqrt(var + 1e-5) * gamma_ref[:] + beta_ref[:]
        xn = jnp.where(xn >= 0, xn, 0.1 * xn)
        o = jnp.sum(xn * w2_ref[:], axis=0, keepdims=True) + b2_ref[:]
        out_ref[:] = jax.nn.sigmoid(o)


def kernel(idsTensor, emb_table, W1, b1, gamma, beta, W2, b2):
    bsz, _ = idsTensor.shape
    n_teams, emb = emb_table.shape
    hid = W1.shape[0]

    n_workers = 32
    chunk = 128
    n_chunks = 2 * bsz // (n_workers * chunk)

    sc_gather = _make_sc_gather(bsz, emb, n_workers, n_chunks, chunk)
    feats = sc_gather(idsTensor.reshape(3 * bsz), emb_table)

    sd = idsTensor[:, 2].reshape(1, bsz)
    w1ab = W1[:, :2 * emb]
    w1c = W1[:, 2 * emb:]
    b1c = b1.reshape(hid, 1)
    gammac = gamma.reshape(hid, 1)
    betac = beta.reshape(hid, 1)
    w2c = W2.reshape(hid, 1)
    b2r = b2.reshape(1, 1)

    bb = 4096
    n_blocks = bsz // bb

    full = lambda shape: pl.BlockSpec(shape, lambda i: (0, 0))
    out = pl.pallas_call(
        functools.partial(_mlp_body, n_blocks, bb),
        grid=(n_blocks,),
        in_specs=[
            pl.BlockSpec((bb, 2 * emb), lambda i: (i, 0)),
            pl.BlockSpec((1, bb), lambda i: (0, i)),
            full((hid, 2 * emb)),
            full((hid, 1)),
            full((hid, 1)),
            full((hid, 1)),
            full((hid, 1)),
            full((hid, 1)),
            full((1, 1)),
        ],
        out_specs=pl.BlockSpec((1, bsz), lambda i: (0, 0)),
        out_shape=jax.ShapeDtypeStruct((1, bsz), jnp.float32),
        scratch_shapes=[pltpu.VMEM((hid, bsz), jnp.float32)],
        compiler_params=pltpu.CompilerParams(
            dimension_semantics=("arbitrary",)),
    )(feats, sd, w1ab, w1c, b1c, gammac, betac, w2c, b2r)
    return out.reshape(bsz, 1)


# speedup vs baseline: 1.2902x; 1.2902x over previous
import functools

import jax
import jax.numpy as jnp
from jax import lax
from jax.experimental import pallas as pl
from jax.experimental.pallas import tpu as pltpu
from jax.experimental.pallas import tpu_sc as plsc


def _make_sc_gather(bsz, emb, n_workers, n_chunks, chunk):
    mesh = plsc.VectorSubcoreMesh(core_axis_name="c", subcore_axis_name="s")
    rows_per_w = n_chunks * chunk
    half = n_workers // 2
    assert rows_per_w * half == bsz

    nbuf = 6
    nl = 16

    @functools.partial(
        pl.kernel,
        mesh=mesh,
        out_type=jax.ShapeDtypeStruct((bsz, 2 * emb), jnp.float32),
        scratch_types=[
            pltpu.VMEM((n_chunks, chunk), jnp.int32),
            [pltpu.VMEM((chunk, emb), jnp.float32) for _ in range(nbuf)],
            [pltpu.SemaphoreType.DMA for _ in range(nbuf)],
            [pltpu.SemaphoreType.DMA for _ in range(nbuf)],
        ],
    )
    def sc_gather(idx_hbm, table_hbm, out_hbm, idx_v, rows, gsems, ssems):
        wid = lax.axis_index("s") * 2 + lax.axis_index("c")
        base = (wid % half) * rows_per_w
        col = (wid // half) * emb
        pltpu.sync_copy(idx_hbm.at[wid], idx_v)
        gh = [None] * nbuf
        sh = [None] * nbuf

        def store_dst(j):
            return out_hbm.at[pl.ds(base + j * chunk, chunk), pl.ds(col, emb)]

        for j in range(min(nbuf, n_chunks)):
            gh[j] = pltpu.async_copy(
                table_hbm.at[idx_v.at[j]], rows[j], gsems[j])
        for j in range(n_chunks):
            s = j % nbuf
            gh[s].wait()
            sh[s] = pltpu.async_copy(rows[s], store_dst(j), ssems[s])
            nxt = j - 1 + nbuf
            if j >= 1 and nxt < n_chunks:
                t = nxt % nbuf
                sh[t].wait()
                gh[t] = pltpu.async_copy(
                    table_hbm.at[idx_v.at[nxt]], rows[t], gsems[t])
        for j in range(max(0, n_chunks - nbuf), n_chunks):
            sh[j % nbuf].wait()

    return sc_gather


def _mlp_body(n_blocks, bb, feats_ref, sd_ref, w1ab_ref, w1c_ref, b1_ref,
              gamma_ref, beta_ref, w2_ref, b2_ref, out_ref, x_scratch):
    i = pl.program_id(0)
    xbt = lax.dot_general(
        w1ab_ref[:], feats_ref[:], (((1,), (1,)), ((), ())),
        preferred_element_type=jnp.float32,
    )
    x_scratch[:, pl.ds(i * bb, bb)] = (
        xbt + sd_ref[:] * w1c_ref[:] + b1_ref[:])

    @pl.when(i == n_blocks - 1)
    def _finish():
        x = x_scratch[:]
        mean = jnp.mean(x, axis=1, keepdims=True)
        var = jnp.mean((x - mean) ** 2, axis=1, keepdims=True)
        xn = (x - mean) * lax.rsqrt(var + 1e-5) * gamma_ref[:] + beta_ref[:]
        xn = jnp.where(xn >= 0, xn, 0.1 * xn)
        o = jnp.sum(xn * w2_ref[:], axis=0, keepdims=True) + b2_ref[:]
        out_ref[:] = jax.nn.sigmoid(o)


def kernel(idsTensor, emb_table, W1, b1, gamma, beta, W2, b2):
    bsz, _ = idsTensor.shape
    n_teams, emb = emb_table.shape
    hid = W1.shape[0]

    n_workers = 32
    chunk = 128
    n_chunks = 2 * bsz // (n_workers * chunk)

    idx = idsTensor[:, :2].astype(jnp.int32).T.reshape(
        n_workers, n_chunks, chunk)
    sc_gather = _make_sc_gather(bsz, emb, n_workers, n_chunks, chunk)
    feats = sc_gather(idx, emb_table)

    sd = idsTensor[:, 2].reshape(1, bsz)
    w1ab = W1[:, :2 * emb]
    w1c = W1[:, 2 * emb:]
    b1c = b1.reshape(hid, 1)
    gammac = gamma.reshape(hid, 1)
    betac = beta.reshape(hid, 1)
    w2c = W2.reshape(hid, 1)
    b2r = b2.reshape(1, 1)

    bb = 4096
    n_blocks = bsz // bb

    full = lambda shape: pl.BlockSpec(shape, lambda i: (0, 0))
    out = pl.pallas_call(
        functools.partial(_mlp_body, n_blocks, bb),
        grid=(n_blocks,),
        in_specs=[
            pl.BlockSpec((bb, 2 * emb), lambda i: (i, 0)),
            pl.BlockSpec((1, bb), lambda i: (0, i)),
            full((hid, 2 * emb)),
            full((hid, 1)),
            full((hid, 1)),
            full((hid, 1)),
            full((hid, 1)),
            full((hid, 1)),
            full((1, 1)),
        ],
        out_specs=pl.BlockSpec((1, bsz), lambda i: (0, 0)),
        out_shape=jax.ShapeDtypeStruct((1, bsz), jnp.float32),
        scratch_shapes=[pltpu.VMEM((hid, bsz), jnp.float32)],
        compiler_params=pltpu.CompilerParams(
            dimension_semantics=("arbitrary",)),
    )(feats, sd, w1ab, w1c, b1c, gammac, betac, w2c, b2r)
    return out.reshape(bsz, 1)
